# baseline (speedup 1.0000x reference)
import functools

import jax
import jax.numpy as jnp
from jax import lax
from jax.experimental import pallas as pl
from jax.experimental.pallas import tpu as pltpu


def _thresh(v):
    return jnp.where(v < 0, 0.0, v)


def _body(xt_ref, u_ref, xs_ref, ys_ref, c_ref, xb_ref, cnt_ref, u2_ref,
          p_ref):
    CH, M, RB = xt_ref.shape
    N = CH * M
    K = u_ref.shape[0]

    carry = _thresh(xt_ref[0])
    xb_ref[0] = carry
    c_ref[0] = carry
    for jj in range(1, CH):
        xj = _thresh(xt_ref[jj])
        xb_ref[jj] = xj
        carry = carry + xj
        c_ref[jj] = carry

    T = c_ref[CH - 1]
    pm = jnp.zeros((RB,), jnp.float32)
    plist = []
    for m in range(M):
        plist.append(pm)
        pm = pm + T[m]
    P = jnp.stack(plist, axis=0)
    total = pm

    c_ref[...] = c_ref[...] + P[None, :, :]

    u2_ref[...] = u_ref[...] * total[None, :]

    def count_body(k, _):
        uk = u2_ref[pl.ds(k, 1), :]
        mask = c_ref[...] <= uk[None, :, :]
        cnt = jnp.sum(mask.astype(jnp.int32), axis=(0, 1))
        cnt_ref[pl.ds(k, 1), :] = cnt[None, :]
        return 0

    lax.fori_loop(0, K, count_body, 0, unroll=False)

    cnt_ref[...] = jnp.minimum(cnt_ref[...], N - 1)

    iota_i = (
        lax.broadcasted_iota(jnp.int32, (CH, M, 1), 0)
        + CH * lax.broadcasted_iota(jnp.int32, (CH, M, 1), 1)
    )

    def prob_body(k, _):
        sk = cnt_ref[pl.ds(k, 1), :]
        eq = iota_i == sk[None, :, :]
        pk = jnp.sum(jnp.where(eq, xb_ref[...], 0.0), axis=(0, 1))
        p_ref[pl.ds(k, 1), :] = pk[None, :]
        return 0

    lax.fori_loop(0, K, prob_body, 0, unroll=False)
    s = cnt_ref[...]
    p = p_ref[...]

    ki = lax.broadcasted_iota(jnp.int32, (K, K, 1), 0)
    kq = lax.broadcasted_iota(jnp.int32, (K, K, 1), 1)
    pk_self = p[:, None, :]
    pk_other = p[None, :, :]
    before = (pk_other > pk_self) | ((pk_other == pk_self) & (kq < ki))
    rank = jnp.sum(before.astype(jnp.int32), axis=1)

    ii = lax.broadcasted_iota(jnp.int32, (K, K, 1), 0)
    eq_r = rank[None, :, :] == ii
    s_sorted = jnp.sum(jnp.where(eq_r, s[None, :, :], 0), axis=1)

    xf = (s_sorted & 63).astype(jnp.float32)
    yf = (s_sorted >> 6).astype(jnp.float32)
    xs_ref[...] = (xf - 32.0) * 0.015625
    ys_ref[...] = (yf - 32.0) * 0.015625


def kernel(heatmap, num_samples):
    b, j, w, h = heatmap.shape
    R = b * j
    N = w * h
    CH = 128
    M = N // CH
    K = 64
    RB = 256 if R % 256 == 0 else R

    xt = heatmap.reshape(R, M, CH).transpose(2, 1, 0)
    u_raw = jax.random.uniform(jax.random.key(42), (R, K),
                               dtype=jnp.float32).T

    grid = (R // RB,)
    xs, ys = pl.pallas_call(
        _body,
        grid=grid,
        in_specs=[
            pl.BlockSpec((CH, M, RB), lambda i: (0, 0, i)),
            pl.BlockSpec((K, RB), lambda i: (0, i)),
        ],
        out_specs=[
            pl.BlockSpec((K, RB), lambda i: (0, i)),
            pl.BlockSpec((K, RB), lambda i: (0, i)),
        ],
        out_shape=[
            jax.ShapeDtypeStruct((K, R), jnp.float32),
            jax.ShapeDtypeStruct((K, R), jnp.float32),
        ],
        scratch_shapes=[
            pltpu.VMEM((CH, M, RB), jnp.float32),
            pltpu.VMEM((CH, M, RB), jnp.float32),
            pltpu.VMEM((K, RB), jnp.int32),
            pltpu.VMEM((K, RB), jnp.float32),
            pltpu.VMEM((K, RB), jnp.float32),
        ],
        compiler_params=pltpu.CompilerParams(
            dimension_semantics=("arbitrary",),
        ),
    )(xt, u_raw)

    xn = xs.reshape(K, b, j)
    yn = ys.reshape(K, b, j)
    out = jnp.stack((xn, yn), axis=-1).transpose(1, 0, 2, 3).reshape(b, K, 2 * j)
    return out


# speedup vs baseline: 1.9121x; 1.9121x over previous
import functools

import jax
import jax.numpy as jnp
from jax import lax
from jax.experimental import pallas as pl
from jax.experimental.pallas import tpu as pltpu
from jax.experimental.pallas import tpu_sc as plsc

_CH = 128
_M = 32
_N = _CH * _M
_K = 64
_NW = 32
_LG2N = 13


def _thresh(v):
    return jnp.where(v < 0, 0.0, v)


def _tc_scan_body(xt_ref, cnat_ref, tot_ref, c_ref):
    CH, M, RB = xt_ref.shape

    carry = _thresh(xt_ref[0])
    c_ref[0] = carry
    for jj in range(1, CH):
        carry = carry + _thresh(xt_ref[jj])
        c_ref[jj] = carry

    T = c_ref[CH - 1]
    pm = jnp.zeros((RB,), jnp.float32)
    plist = []
    for m in range(M):
        plist.append(pm)
        pm = pm + T[m]
    P = jnp.stack(plist, axis=0)

    c_ref[...] = c_ref[...] + P[None, :, :]
    tot_ref[...] = pm[None, :]

    for m in range(M):
        cnat_ref[:, m, :] = jnp.transpose(c_ref[:, m, :])


def _sc_search_body(rpw, c_hbm, u_hbm, tot_hbm, flat_hbm, s_hbm, p_hbm,
                    cbuf, fbuf, ubuf, tbuf, sall, pall, sem, semf):
    wid = lax.axis_index("s") * 2 + lax.axis_index("c")
    base = wid * rpw

    pltpu.sync_copy(u_hbm.at[pl.ds(base * _K, rpw * _K)], ubuf)
    pltpu.sync_copy(tot_hbm.at[pl.ds(base, rpw)], tbuf)

    def row_body(r, _):
        copy_f = pltpu.async_copy(flat_hbm.at[base + r], fbuf, semf)
        pltpu.async_copy(c_hbm.at[base + r], cbuf, sem).wait()
        t = plsc.load_gather(tbuf, [jnp.full((16,), r, jnp.int32)])

        for g in range(_K // 16):
            uraw = ubuf[pl.ds(r * _K + g * 16, 16)]
            u2 = uraw * t
            lo = jnp.zeros((16,), jnp.int32)
            hi = jnp.full((16,), _N, jnp.int32)
            for _step in range(_LG2N):
                mid = jnp.minimum(jnp.right_shift(lo + hi, 1), _N - 1)
                v = plsc.load_gather(cbuf, [mid])
                pred = v <= u2
                lo = jnp.where(pred, mid + 1, lo)
                hi = jnp.where(pred, hi, mid)
            s = jnp.minimum(lo, _N - 1)
            sall[pl.ds(r * _K + g * 16, 16)] = s
            if g == 0:
                copy_f.wait()
            vals = plsc.load_gather(fbuf, [s])
            pall[pl.ds(r * _K + g * 16, 16)] = _thresh(vals)
        return 0

    lax.fori_loop(0, rpw, row_body, 0, unroll=False)

    pltpu.sync_copy(sall, s_hbm.at[pl.ds(base * _K, rpw * _K)])
    pltpu.sync_copy(pall, p_hbm.at[pl.ds(base * _K, rpw * _K)])


def _tc_sort_body(s_ref, p_ref, xs_ref, ys_ref):
    K, RB = s_ref.shape
    s = s_ref[...]
    p = p_ref[...]

    ki = lax.broadcasted_iota(jnp.int32, (K, K, 1), 0)
    kq = lax.broadcasted_iota(jnp.int32, (K, K, 1), 1)
    pk_self = p[:, None, :]
    pk_other = p[None, :, :]
    before = (pk_other > pk_self) | ((pk_other == pk_self) & (kq < ki))
    rank = jnp.sum(before.astype(jnp.int32), axis=1)

    ii = lax.broadcasted_iota(jnp.int32, (K, K, 1), 0)
    eq_r = rank[None, :, :] == ii
    s_sorted = jnp.sum(jnp.where(eq_r, s[None, :, :], 0), axis=1)

    xf = (s_sorted & 63).astype(jnp.float32)
    yf = (s_sorted >> 6).astype(jnp.float32)
    xs_ref[...] = (xf - 32.0) * 0.015625
    ys_ref[...] = (yf - 32.0) * 0.015625


def kernel(heatmap, num_samples):
    b, j, w, h = heatmap.shape
    R = b * j
    RB = 256 if R % 256 == 0 else R

    xt = heatmap.reshape(R, _M, _CH).transpose(2, 1, 0)
    u_raw = jax.random.uniform(jax.random.key(42), (R, _K),
                               dtype=jnp.float32)

    grid = (R // RB,)
    c_nat, tot = pl.pallas_call(
        _tc_scan_body,
        grid=grid,
        in_specs=[pl.BlockSpec((_CH, _M, RB), lambda i: (0, 0, i))],
        out_specs=[
            pl.BlockSpec((RB, _M, _CH), lambda i: (i, 0, 0)),
            pl.BlockSpec((1, RB), lambda i: (0, i)),
        ],
        out_shape=[
            jax.ShapeDtypeStruct((R, _M, _CH), jnp.float32),
            jax.ShapeDtypeStruct((1, R), jnp.float32),
        ],
        scratch_shapes=[pltpu.VMEM((_CH, _M, RB), jnp.float32)],
        compiler_params=pltpu.CompilerParams(
            dimension_semantics=("arbitrary",),
        ),
    )(xt)

    rpw = R // _NW
    mesh = plsc.VectorSubcoreMesh(core_axis_name="c", subcore_axis_name="s")
    sc = pl.kernel(
        functools.partial(_sc_search_body, rpw),
        out_type=[
            jax.ShapeDtypeStruct((R * _K,), jnp.int32),
            jax.ShapeDtypeStruct((R * _K,), jnp.float32),
        ],
        mesh=mesh,
        scratch_types=[
            pltpu.VMEM((_N,), jnp.float32),
            pltpu.VMEM((_N,), jnp.float32),
            pltpu.VMEM((rpw * _K,), jnp.float32),
            pltpu.VMEM((rpw,), jnp.float32),
            pltpu.VMEM((rpw * _K,), jnp.int32),
            pltpu.VMEM((rpw * _K,), jnp.float32),
            pltpu.SemaphoreType.DMA,
            pltpu.SemaphoreType.DMA,
        ],
        compiler_params=pltpu.CompilerParams(needs_layout_passes=False),
    )
    s_flat, p_flat = sc(
        c_nat.reshape(R, _N),
        u_raw.reshape(R * _K),
        tot.reshape(R),
        heatmap.reshape(R, _N),
    )

    s_t = s_flat.reshape(R, _K).T
    p_t = p_flat.reshape(R, _K).T

    xs, ys = pl.pallas_call(
        _tc_sort_body,
        grid=grid,
        in_specs=[
            pl.BlockSpec((_K, RB), lambda i: (0, i)),
            pl.BlockSpec((_K, RB), lambda i: (0, i)),
        ],
        out_specs=[
            pl.BlockSpec((_K, RB), lambda i: (0, i)),
            pl.BlockSpec((_K, RB), lambda i: (0, i)),
        ],
        out_shape=[
            jax.ShapeDtypeStruct((_K, R), jnp.float32),
            jax.ShapeDtypeStruct((_K, R), jnp.float32),
        ],
        compiler_params=pltpu.CompilerParams(
            dimension_semantics=("arbitrary",),
        ),
    )(s_t, p_t)

    xn = xs.reshape(_K, b, j)
    yn = ys.reshape(_K, b, j)
    out = jnp.stack((xn, yn), axis=-1).transpose(1, 0, 2, 3).reshape(
        b, _K, 2 * j)
    return out


# speedup vs baseline: 2.0585x; 1.0766x over previous
import functools

import jax
import jax.numpy as jnp
from jax import lax
from jax.experimental import pallas as pl
from jax.experimental.pallas import tpu as pltpu
from jax.experimental.pallas import tpu_sc as plsc

_CH = 128
_M = 32
_N = _CH * _M
_K = 64
_NW = 32
_LG2N = 13
_G = 8


def _thresh(v):
    return jnp.where(v < 0, 0.0, v)


def _tc_scan_body(x_ref, c8_ref, f8_ref, tot_ref, xt_ref, c_ref):
    RB = x_ref.shape[0]

    for m in range(_M):
        xt_ref[:, m, :] = jnp.transpose(x_ref[:, m, :])

    carry = _thresh(xt_ref[0])
    c_ref[0] = carry
    for jj in range(1, _CH):
        carry = carry + _thresh(xt_ref[jj])
        c_ref[jj] = carry

    T = c_ref[_CH - 1]
    pm = jnp.zeros((RB,), jnp.float32)
    plist = []
    for m in range(_M):
        plist.append(pm)
        pm = pm + T[m]
    P = jnp.stack(plist, axis=0)

    c_ref[...] = c_ref[...] + P[None, :, :]
    tot_ref[...] = pm[None, :]

    for m in range(_M):
        c8_ref[:, m, :, :] = jnp.transpose(c_ref[:, m, :]).reshape(
            RB // _G, _G, _CH)
        f8_ref[:, m, :, :] = jnp.transpose(_thresh(xt_ref[:, m, :])).reshape(
            RB // _G, _G, _CH)


def _sc_search_body(rpw, c8_hbm, u_hbm, tot_hbm, f8_hbm, s_hbm, p_hbm,
                    cbuf, fbuf, ubuf, tbuf, sall, pall, semc, semf):
    ng = rpw // _G
    wid = lax.axis_index("s") * 2 + lax.axis_index("c")
    base = wid * rpw
    gbase = wid * ng

    pltpu.sync_copy(u_hbm.at[pl.ds(base * _K, rpw * _K)], ubuf)
    pltpu.sync_copy(tot_hbm.at[pl.ds(base, rpw)], tbuf)

    pltpu.async_copy(c8_hbm.at[gbase], cbuf.at[0], semc.at[0])

    def group_body(g, _):
        buf = lax.rem(g, 2)
        nbuf = 1 - buf
        @pl.when(g + 1 < ng)
        def _():
            pltpu.async_copy(c8_hbm.at[gbase + g + 1], cbuf.at[nbuf],
                             semc.at[nbuf])
        fcopy = pltpu.async_copy(f8_hbm.at[gbase + g], fbuf, semf)
        pltpu.make_async_copy(c8_hbm.at[gbase + g], cbuf.at[buf],
                              semc.at[buf]).wait()

        buf16 = jnp.full((16,), buf, jnp.int32)

        def row_body(rlo, _2):
            r = g * _G + rlo
            t = plsc.load_gather(tbuf, [jnp.full((16,), r, jnp.int32)])
            off = rlo * _CH
            for gk in range(_K // 16):
                uraw = ubuf[pl.ds(r * _K + gk * 16, 16)]
                u2 = uraw * t
                lo = jnp.zeros((16,), jnp.int32)
                hi = jnp.full((16,), _N, jnp.int32)
                for _step in range(_LG2N):
                    mid = jnp.minimum(jnp.right_shift(lo + hi, 1), _N - 1)
                    adr = ((mid >> 7) << 10) + off + (mid & 127)
                    v = plsc.load_gather(cbuf, [buf16, adr])
                    pred = v <= u2
                    lo = jnp.where(pred, mid + 1, lo)
                    hi = jnp.where(pred, hi, mid)
                s = jnp.minimum(lo, _N - 1)
                sall[pl.ds(r * _K + gk * 16, 16)] = s
            return 0

        lax.fori_loop(0, _G, row_body, 0, unroll=False)

        fcopy.wait()

        def prob_body(rlo, _2):
            r = g * _G + rlo
            off = rlo * _CH
            for gk in range(_K // 16):
                s = sall[pl.ds(r * _K + gk * 16, 16)]
                adr = ((s >> 7) << 10) + off + (s & 127)
                vals = plsc.load_gather(fbuf, [adr])
                pall[pl.ds(r * _K + gk * 16, 16)] = vals
            return 0

        lax.fori_loop(0, _G, prob_body, 0, unroll=False)
        return 0

    lax.fori_loop(0, ng, group_body, 0, unroll=False)

    pltpu.sync_copy(sall, s_hbm.at[pl.ds(base * _K, rpw * _K)])
    pltpu.sync_copy(pall, p_hbm.at[pl.ds(base * _K, rpw * _K)])


def _tc_sort_body(s_ref, p_ref, xs_ref, ys_ref):
    K, RB = s_ref.shape
    s = s_ref[...]
    p = p_ref[...]

    ki = lax.broadcasted_iota(jnp.int32, (K, K, 1), 0)
    kq = lax.broadcasted_iota(jnp.int32, (K, K, 1), 1)
    pk_self = p[:, None, :]
    pk_other = p[None, :, :]
    before = (pk_other > pk_self) | ((pk_other == pk_self) & (kq < ki))
    rank = jnp.sum(before.astype(jnp.int32), axis=1)

    ii = lax.broadcasted_iota(jnp.int32, (K, K, 1), 0)
    eq_r = rank[None, :, :] == ii
    s_sorted = jnp.sum(jnp.where(eq_r, s[None, :, :], 0), axis=1)

    xf = (s_sorted & 63).astype(jnp.float32)
    yf = (s_sorted >> 6).astype(jnp.float32)
    xs_ref[...] = (xf - 32.0) * 0.015625
    ys_ref[...] = (yf - 32.0) * 0.015625


def kernel(heatmap, num_samples):
    b, j, w, h = heatmap.shape
    R = b * j
    RB = 256 if R % 256 == 0 else R

    u_raw = jax.random.uniform(jax.random.key(42), (R, _K),
                               dtype=jnp.float32)

    grid = (R // RB,)
    c8, f8, tot = pl.pallas_call(
        _tc_scan_body,
        grid=grid,
        in_specs=[pl.BlockSpec((RB, _M, _CH), lambda i: (i, 0, 0))],
        out_specs=[
            pl.BlockSpec((RB // _G, _M, _G, _CH), lambda i: (i, 0, 0, 0)),
            pl.BlockSpec((RB // _G, _M, _G, _CH), lambda i: (i, 0, 0, 0)),
            pl.BlockSpec((1, RB), lambda i: (0, i)),
        ],
        out_shape=[
            jax.ShapeDtypeStruct((R // _G, _M, _G, _CH), jnp.float32),
            jax.ShapeDtypeStruct((R // _G, _M, _G, _CH), jnp.float32),
            jax.ShapeDtypeStruct((1, R), jnp.float32),
        ],
        scratch_shapes=[
            pltpu.VMEM((_CH, _M, RB), jnp.float32),
            pltpu.VMEM((_CH, _M, RB), jnp.float32),
        ],
        compiler_params=pltpu.CompilerParams(
            dimension_semantics=("arbitrary",),
        ),
    )(heatmap.reshape(R, _M, _CH))

    rpw = R // _NW
    mesh = plsc.VectorSubcoreMesh(core_axis_name="c", subcore_axis_name="s")
    sc = pl.kernel(
        functools.partial(_sc_search_body, rpw),
        out_type=[
            jax.ShapeDtypeStruct((R * _K,), jnp.int32),
            jax.ShapeDtypeStruct((R * _K,), jnp.float32),
        ],
        mesh=mesh,
        scratch_types=[
            pltpu.VMEM((2, _M * _G * _CH), jnp.float32),
            pltpu.VMEM((_M * _G * _CH,), jnp.float32),
            pltpu.VMEM((rpw * _K,), jnp.float32),
            pltpu.VMEM((rpw,), jnp.float32),
            pltpu.VMEM((rpw * _K,), jnp.int32),
            pltpu.VMEM((rpw * _K,), jnp.float32),
            pltpu.SemaphoreType.DMA((2,)),
            pltpu.SemaphoreType.DMA,
        ],
        compiler_params=pltpu.CompilerParams(needs_layout_passes=False),
    )
    s_flat, p_flat = sc(
        c8.reshape(R // _G, _M * _G * _CH),
        u_raw.reshape(R * _K),
        tot.reshape(R),
        f8.reshape(R // _G, _M * _G * _CH),
    )

    s_t = s_flat.reshape(R, _K).T
    p_t = p_flat.reshape(R, _K).T

    xs, ys = pl.pallas_call(
        _tc_sort_body,
        grid=grid,
        in_specs=[
            pl.BlockSpec((_K, RB), lambda i: (0, i)),
            pl.BlockSpec((_K, RB), lambda i: (0, i)),
        ],
        out_specs=[
            pl.BlockSpec((_K, RB), lambda i: (0, i)),
            pl.BlockSpec((_K, RB), lambda i: (0, i)),
        ],
        out_shape=[
            jax.ShapeDtypeStruct((_K, R), jnp.float32),
            jax.ShapeDtypeStruct((_K, R), jnp.float32),
        ],
        compiler_params=pltpu.CompilerParams(
            dimension_semantics=("arbitrary",),
        ),
    )(s_t, p_t)

    xn = xs.reshape(_K, b, j)
    yn = ys.reshape(_K, b, j)
    out = jnp.stack((xn, yn), axis=-1).transpose(1, 0, 2, 3).reshape(
        b, _K, 2 * j)
    return out


# speedup vs baseline: 2.4009x; 1.1663x over previous
import functools

import jax
import jax.numpy as jnp
from jax import lax
from jax.experimental import pallas as pl
from jax.experimental.pallas import tpu as pltpu
from jax.experimental.pallas import tpu_sc as plsc

_CH = 128
_M = 32
_N = _CH * _M
_K = 64
_NW = 32
_LG2N = 13
_G = 8


def _thresh(v):
    return jnp.where(v < 0, 0.0, v)


def _tc_scan_body(x_ref, c8_ref, f8_ref, tot_ref, xt_ref, c_ref):
    RB = x_ref.shape[0]

    for m in range(_M):
        xt_ref[:, m, :] = jnp.transpose(x_ref[:, m, :])

    carry = _thresh(xt_ref[0])
    c_ref[0] = carry
    for jj in range(1, _CH):
        carry = carry + _thresh(xt_ref[jj])
        c_ref[jj] = carry

    T = c_ref[_CH - 1]
    pm = jnp.zeros((RB,), jnp.float32)
    plist = []
    for m in range(_M):
        plist.append(pm)
        pm = pm + T[m]
    P = jnp.stack(plist, axis=0)

    c_ref[...] = c_ref[...] + P[None, :, :]
    tot_ref[...] = pm[None, :]

    for m in range(_M):
        c8_ref[:, m, :, :] = jnp.transpose(c_ref[:, m, :]).reshape(
            RB // _G, _G, _CH)
        f8_ref[:, m, :, :] = jnp.transpose(_thresh(xt_ref[:, m, :])).reshape(
            RB // _G, _G, _CH)


def _sc_search_body(rpw, c8_hbm, u_hbm, tot_hbm, f8_hbm, s_hbm, p_hbm,
                    cbuf, fbuf, ubuf, tbuf, sall, pall, semc, semf):
    ng = rpw // _G
    wid = lax.axis_index("s") * 2 + lax.axis_index("c")
    base = wid * rpw
    gbase = wid * ng

    pltpu.sync_copy(u_hbm.at[pl.ds(base * _K, rpw * _K)], ubuf)
    pltpu.sync_copy(tot_hbm.at[pl.ds(base, rpw)], tbuf)

    pltpu.async_copy(c8_hbm.at[gbase], cbuf.at[0], semc.at[0])

    def group_body(g, _):
        buf = lax.rem(g, 2)
        nbuf = 1 - buf
        @pl.when(g + 1 < ng)
        def _():
            pltpu.async_copy(c8_hbm.at[gbase + g + 1], cbuf.at[nbuf],
                             semc.at[nbuf])
        fcopy = pltpu.async_copy(f8_hbm.at[gbase + g], fbuf, semf)
        pltpu.make_async_copy(c8_hbm.at[gbase + g], cbuf.at[buf],
                              semc.at[buf]).wait()

        buf16 = jnp.full((16,), buf, jnp.int32)

        def row_body(rlo, _2):
            r = g * _G + rlo
            t = plsc.load_gather(tbuf, [jnp.full((16,), r, jnp.int32)])
            off = rlo * _CH
            for gk in range(_K // 16):
                uraw = ubuf[pl.ds(r * _K + gk * 16, 16)]
                u2 = uraw * t
                lo = jnp.zeros((16,), jnp.int32)
                hi = jnp.full((16,), _N, jnp.int32)
                for _step in range(_LG2N):
                    mid = jnp.minimum(jnp.right_shift(lo + hi, 1), _N - 1)
                    adr = ((mid >> 7) << 10) + off + (mid & 127)
                    v = plsc.load_gather(cbuf, [buf16, adr])
                    pred = v <= u2
                    lo = jnp.where(pred, mid + 1, lo)
                    hi = jnp.where(pred, hi, mid)
                s = jnp.minimum(lo, _N - 1)
                sall[pl.ds(r * _K + gk * 16, 16)] = s
            return 0

        lax.fori_loop(0, _G, row_body, 0, unroll=False)

        fcopy.wait()

        def prob_body(rlo, _2):
            r = g * _G + rlo
            off = rlo * _CH
            for gk in range(_K // 16):
                s = sall[pl.ds(r * _K + gk * 16, 16)]
                adr = ((s >> 7) << 10) + off + (s & 127)
                vals = plsc.load_gather(fbuf, [adr])
                pall[pl.ds(r * _K + gk * 16, 16)] = vals
            return 0

        lax.fori_loop(0, _G, prob_body, 0, unroll=False)
        return 0

    lax.fori_loop(0, ng, group_body, 0, unroll=False)

    pltpu.sync_copy(sall, s_hbm.at[pl.ds(base * _K, rpw * _K)])
    pltpu.sync_copy(pall, p_hbm.at[pl.ds(base * _K, rpw * _K)])


def _tc_sort_body(s_ref, p_ref, xs_ref, ys_ref):
    RB, K = s_ref.shape
    s = jnp.transpose(s_ref[...])
    p = jnp.transpose(p_ref[...])

    ki = lax.broadcasted_iota(jnp.int32, (K, 1), 0)
    rank = jnp.zeros(s.shape, jnp.int32)
    for kq in range(K):
        pq = p[kq][None, :]
        before = (pq > p) | ((pq == p) & (kq < ki))
        rank = rank + before.astype(jnp.int32)

    s_sorted = jnp.zeros(s.shape, jnp.int32)
    for kq in range(K):
        hit = rank[kq][None, :] == ki
        s_sorted = s_sorted + jnp.where(hit, s[kq][None, :], 0)

    xf = (s_sorted & 63).astype(jnp.float32)
    yf = (s_sorted >> 6).astype(jnp.float32)
    xs_ref[...] = (xf - 32.0) * 0.015625
    ys_ref[...] = (yf - 32.0) * 0.015625


def kernel(heatmap, num_samples):
    b, j, w, h = heatmap.shape
    R = b * j
    RB = 256 if R % 256 == 0 else R

    u_raw = jax.random.uniform(jax.random.key(42), (R, _K),
                               dtype=jnp.float32)

    grid = (R // RB,)
    c8, f8, tot = pl.pallas_call(
        _tc_scan_body,
        grid=grid,
        in_specs=[pl.BlockSpec((RB, _M, _CH), lambda i: (i, 0, 0))],
        out_specs=[
            pl.BlockSpec((RB // _G, _M, _G, _CH), lambda i: (i, 0, 0, 0)),
            pl.BlockSpec((RB // _G, _M, _G, _CH), lambda i: (i, 0, 0, 0)),
            pl.BlockSpec((1, RB), lambda i: (0, i)),
        ],
        out_shape=[
            jax.ShapeDtypeStruct((R // _G, _M, _G, _CH), jnp.float32),
            jax.ShapeDtypeStruct((R // _G, _M, _G, _CH), jnp.float32),
            jax.ShapeDtypeStruct((1, R), jnp.float32),
        ],
        scratch_shapes=[
            pltpu.VMEM((_CH, _M, RB), jnp.float32),
            pltpu.VMEM((_CH, _M, RB), jnp.float32),
        ],
        compiler_params=pltpu.CompilerParams(
            dimension_semantics=("arbitrary",),
        ),
    )(heatmap.reshape(R, _M, _CH))

    rpw = R // _NW
    mesh = plsc.VectorSubcoreMesh(core_axis_name="c", subcore_axis_name="s")
    sc = pl.kernel(
        functools.partial(_sc_search_body, rpw),
        out_type=[
            jax.ShapeDtypeStruct((R * _K,), jnp.int32),
            jax.ShapeDtypeStruct((R * _K,), jnp.float32),
        ],
        mesh=mesh,
        scratch_types=[
            pltpu.VMEM((2, _M * _G * _CH), jnp.float32),
            pltpu.VMEM((_M * _G * _CH,), jnp.float32),
            pltpu.VMEM((rpw * _K,), jnp.float32),
            pltpu.VMEM((rpw,), jnp.float32),
            pltpu.VMEM((rpw * _K,), jnp.int32),
            pltpu.VMEM((rpw * _K,), jnp.float32),
            pltpu.SemaphoreType.DMA((2,)),
            pltpu.SemaphoreType.DMA,
        ],
        compiler_params=pltpu.CompilerParams(needs_layout_passes=False),
    )
    s_flat, p_flat = sc(
        c8.reshape(R // _G, _M * _G * _CH),
        u_raw.reshape(R * _K),
        tot.reshape(R),
        f8.reshape(R // _G, _M * _G * _CH),
    )

    xs, ys = pl.pallas_call(
        _tc_sort_body,
        grid=grid,
        in_specs=[
            pl.BlockSpec((RB, _K), lambda i: (i, 0)),
            pl.BlockSpec((RB, _K), lambda i: (i, 0)),
        ],
        out_specs=[
            pl.BlockSpec((_K, RB), lambda i: (0, i)),
            pl.BlockSpec((_K, RB), lambda i: (0, i)),
        ],
        out_shape=[
            jax.ShapeDtypeStruct((_K, R), jnp.float32),
            jax.ShapeDtypeStruct((_K, R), jnp.float32),
        ],
        compiler_params=pltpu.CompilerParams(
            dimension_semantics=("arbitrary",),
        ),
    )(s_flat.reshape(R, _K), p_flat.reshape(R, _K))

    xn = xs.reshape(_K, b, j)
    yn = ys.reshape(_K, b, j)
    out = jnp.stack((xn, yn), axis=-1).transpose(1, 0, 2, 3).reshape(
        b, _K, 2 * j)
    return out
